# Initial kernel scaffold; baseline (speedup 1.0000x reference)
import functools

import jax
import jax.numpy as jnp
from jax import lax
from jax.experimental import pallas as pl
from jax.experimental.pallas import tpu as pltpu
from jax.experimental.pallas import tpu_sc as plsc

N_NODES = 10000
N_EDGES = 320000
D_IN = 128
HID = 128

NC = 2
NS = 16
NW = NC * NS

CHUNK = 80
EPW = N_EDGES // NW
KCH = EPW // CHUNK
ROWS_T = 624
ROWS_LAST = N_NODES - 15 * ROWS_T

NB = 5
BLK = N_NODES // NB
NUM_GRAPHS = 64


def _make_sc_agg(feat, with_deg):
    mesh = plsc.VectorSubcoreMesh(core_axis_name="c", subcore_axis_name="s")
    out_type = [jax.ShapeDtypeStruct((NC, N_NODES, feat), jnp.float32)]
    scratch = [
        pltpu.VMEM((KCH, CHUNK), jnp.int32),
        pltpu.VMEM((KCH, CHUNK), jnp.int32),
        pltpu.VMEM((CHUNK, feat), jnp.float32),
        pltpu.VMEM_SHARED((N_NODES, feat), jnp.float32),
        pltpu.SemaphoreType.DMA,
    ]
    if with_deg:
        out_type.append(jax.ShapeDtypeStruct((NC, N_NODES, 16), jnp.float32))
        scratch += [
            pltpu.VMEM((CHUNK, 16), jnp.float32),
            pltpu.VMEM_SHARED((N_NODES, 16), jnp.float32),
        ]

    def body(x_hbm, src_hbm, dst_hbm, zrow_hbm, zdeg_hbm, ones_hbm, *rest):
        if with_deg:
            acc_out, deg_out, srcv, dstv, rows, acc_sh, sem, ones, deg_sh = rest
        else:
            acc_out, srcv, dstv, rows, acc_sh, sem = rest
        c = lax.axis_index("c")
        s = lax.axis_index("s")
        start = s * ROWS_T

        @pl.when(s < NS - 1)
        def _():
            pltpu.sync_copy(zrow_hbm.at[pl.ds(0, ROWS_T)],
                            acc_sh.at[pl.ds(start, ROWS_T)])

        @pl.when(s == NS - 1)
        def _():
            pltpu.sync_copy(zrow_hbm, acc_sh.at[pl.ds(start, ROWS_LAST)])

        if with_deg:
            @pl.when(s < NS - 1)
            def _():
                pltpu.sync_copy(zdeg_hbm.at[pl.ds(0, ROWS_T)],
                                deg_sh.at[pl.ds(start, ROWS_T)])

            @pl.when(s == NS - 1)
            def _():
                pltpu.sync_copy(zdeg_hbm, deg_sh.at[pl.ds(start, ROWS_LAST)])

            pltpu.sync_copy(ones_hbm, ones)

        wid = c * NS + s
        pltpu.sync_copy(src_hbm.at[pl.ds(wid * KCH, KCH)], srcv)
        pltpu.sync_copy(dst_hbm.at[pl.ds(wid * KCH, KCH)], dstv)

        plsc.subcore_barrier()

        def step(k, carry):
            pltpu.async_copy(x_hbm.at[srcv.at[k]], rows, sem).wait()
            pltpu.sync_copy(rows, acc_sh.at[dstv.at[k]], add=True)
            if with_deg:
                pltpu.sync_copy(ones, deg_sh.at[dstv.at[k]], add=True)
            return carry

        lax.fori_loop(0, KCH, step, 0)

        plsc.subcore_barrier()

        @pl.when(s < NS - 1)
        def _():
            pltpu.sync_copy(acc_sh.at[pl.ds(start, ROWS_T)],
                            acc_out.at[c, pl.ds(start, ROWS_T)])

        @pl.when(s == NS - 1)
        def _():
            pltpu.sync_copy(acc_sh.at[pl.ds(start, ROWS_LAST)],
                            acc_out.at[c, pl.ds(start, ROWS_LAST)])

        if with_deg:
            @pl.when(s < NS - 1)
            def _():
                pltpu.sync_copy(deg_sh.at[pl.ds(start, ROWS_T)],
                                deg_out.at[c, pl.ds(start, ROWS_T)])

            @pl.when(s == NS - 1)
            def _():
                pltpu.sync_copy(deg_sh.at[pl.ds(start, ROWS_LAST)],
                                deg_out.at[c, pl.ds(start, ROWS_LAST)])

    return pl.kernel(body, out_type=tuple(out_type), mesh=mesh,
                     scratch_types=scratch)


_sc_agg_l1 = _make_sc_agg(D_IN, with_deg=True)
_sc_agg_l2 = _make_sc_agg(HID, with_deg=False)


def _dot_t(a, w):
    return lax.dot_general(a, w, (((1,), (1,)), ((), ())),
                           preferred_element_type=jnp.float32)


def _l2n(v):
    n = jnp.sqrt(jnp.sum(v * v, axis=1, keepdims=True))
    return v / jnp.maximum(n, 1e-12)


def _tc1_body(x_ref, acc_ref, deg_ref, wl_ref, bl_ref, wr_ref, h_ref):
    a = acc_ref[0] + acc_ref[1]
    dg = deg_ref[0, :, 0:1] + deg_ref[1, :, 0:1]
    agg = a / jnp.maximum(dg, 1.0)
    out = _dot_t(agg, wl_ref[...]) + bl_ref[...] + _dot_t(x_ref[...], wr_ref[...])
    h_ref[...] = jnp.maximum(_l2n(out), 0.0)


def _tc2_body(h_ref, acc_ref, deg_ref, batch_ref, wl_ref, bl_ref, wr_ref,
              wlin_ref, blin_ref, out_ref, gsum, gcnt):
    i = pl.program_id(0)

    @pl.when(i == 0)
    def _():
        gsum[...] = jnp.zeros_like(gsum)
        gcnt[...] = jnp.zeros_like(gcnt)

    a = acc_ref[0] + acc_ref[1]
    dg = deg_ref[0, :, 0:1] + deg_ref[1, :, 0:1]
    agg = a / jnp.maximum(dg, 1.0)
    out = _dot_t(agg, wl_ref[...]) + bl_ref[...] + _dot_t(h_ref[...], wr_ref[...])
    h2 = _l2n(out)

    b = batch_ref[0, 0]
    gid = lax.broadcasted_iota(jnp.int32, (NUM_GRAPHS, BLK), 0)
    mask = (b[None, :] == gid).astype(jnp.float32)
    gsum[...] += jnp.dot(mask, h2, preferred_element_type=jnp.float32)
    gcnt[...] += jnp.sum(mask, axis=1, keepdims=True)

    @pl.when(i == NB - 1)
    def _():
        g = gsum[...] / jnp.maximum(gcnt[...], 1.0)
        logits = _dot_t(g, wlin_ref[...]) + blin_ref[...]
        out_ref[...] = jax.nn.sigmoid(logits)


_tc1 = pl.pallas_call(
    _tc1_body,
    grid=(NB,),
    in_specs=[
        pl.BlockSpec((BLK, D_IN), lambda i: (i, 0)),
        pl.BlockSpec((NC, BLK, D_IN), lambda i: (0, i, 0)),
        pl.BlockSpec((NC, BLK, 16), lambda i: (0, i, 0)),
        pl.BlockSpec((HID, D_IN), lambda i: (0, 0)),
        pl.BlockSpec((1, HID), lambda i: (0, 0)),
        pl.BlockSpec((HID, D_IN), lambda i: (0, 0)),
    ],
    out_specs=pl.BlockSpec((BLK, HID), lambda i: (i, 0)),
    out_shape=jax.ShapeDtypeStruct((N_NODES, HID), jnp.float32),
)

_tc2 = pl.pallas_call(
    _tc2_body,
    grid=(NB,),
    in_specs=[
        pl.BlockSpec((BLK, HID), lambda i: (i, 0)),
        pl.BlockSpec((NC, BLK, HID), lambda i: (0, i, 0)),
        pl.BlockSpec((NC, BLK, 16), lambda i: (0, i, 0)),
        pl.BlockSpec((1, 1, BLK), lambda i: (i, 0, 0)),
        pl.BlockSpec((2 * HID, HID), lambda i: (0, 0)),
        pl.BlockSpec((1, 2 * HID), lambda i: (0, 0)),
        pl.BlockSpec((2 * HID, HID), lambda i: (0, 0)),
        pl.BlockSpec((1, 2 * HID), lambda i: (0, 0)),
        pl.BlockSpec((1, 1), lambda i: (0, 0)),
    ],
    out_specs=pl.BlockSpec((NUM_GRAPHS, 1), lambda i: (0, 0)),
    out_shape=jax.ShapeDtypeStruct((NUM_GRAPHS, 1), jnp.float32),
    scratch_shapes=[
        pltpu.VMEM((NUM_GRAPHS, 2 * HID), jnp.float32),
        pltpu.VMEM((NUM_GRAPHS, 1), jnp.float32),
    ],
)


def kernel(x, edge_index, batch, Wl1, bl1, Wr1, Wl2, bl2, Wr2, Wlin, blin):
    src = edge_index[0].astype(jnp.int32).reshape(NW * KCH, CHUNK)
    dst = edge_index[1].astype(jnp.int32).reshape(NW * KCH, CHUNK)
    batch3 = batch.astype(jnp.int32).reshape(NB, 1, BLK)
    zrow = jnp.zeros((ROWS_LAST, D_IN), jnp.float32)
    zdeg = jnp.zeros((ROWS_LAST, 16), jnp.float32)
    ones = jnp.ones((CHUNK, 16), jnp.float32)

    acc1, deg = _sc_agg_l1(x, src, dst, zrow, zdeg, ones)
    h = _tc1(x, acc1, deg, Wl1, bl1.reshape(1, HID), Wr1)
    (acc2,) = _sc_agg_l2(h, src, dst, zrow, zdeg, ones)
    out = _tc2(h, acc2, deg, batch3, Wl2, bl2.reshape(1, 2 * HID), Wr2,
               Wlin, blin.reshape(1, 1))
    return out


# speedup vs baseline: 7.5227x; 7.5227x over previous
import functools

import jax
import jax.numpy as jnp
from jax import lax
from jax.experimental import pallas as pl
from jax.experimental.pallas import tpu as pltpu
from jax.experimental.pallas import tpu_sc as plsc

N_NODES = 10000
N_EDGES = 320000
D_IN = 128
HID = 128

NC = 2
NS = 16
NW = NC * NS

CHUNK = 125
EPW = N_EDGES // NW
KCH = EPW // CHUNK
GRP = 8
NGRP = KCH // GRP
ROWS_T = 624
ROWS_LAST = N_NODES - 15 * ROWS_T
BCH = 16

NB = 5
BLK = N_NODES // NB
NUM_GRAPHS = 64


def _tile_rows(s):
    start = s * ROWS_T
    nch = (jnp.where(s == NS - 1, ROWS_LAST, ROWS_T) // BCH).astype(jnp.int32)
    return start, nch


def _make_sc_agg(feat):
    mesh = plsc.VectorSubcoreMesh(core_axis_name="c", subcore_axis_name="s")

    @functools.partial(
        pl.kernel, mesh=mesh,
        out_type=jax.ShapeDtypeStruct((NC, N_NODES, feat), jnp.float32),
        scratch_types=[
            pltpu.VMEM((GRP, CHUNK), jnp.int32),
            pltpu.VMEM((GRP, CHUNK), jnp.int32),
            pltpu.VMEM((CHUNK, feat), jnp.float32),
            pltpu.VMEM((BCH, feat), jnp.float32),
            pltpu.VMEM_SHARED((N_NODES, feat), jnp.float32),
            pltpu.SemaphoreType.DMA,
        ],
    )
    def body(x_hbm, src_hbm, dst_hbm, zrow_hbm, acc_out,
             srcv, dstv, rows, bounce, acc_sh, sem):
        c = lax.axis_index("c")
        s = lax.axis_index("s")
        start, nch = _tile_rows(s)
        wid = c * NS + s

        pltpu.sync_copy(zrow_hbm, bounce)

        def zstep(t, carry):
            pltpu.sync_copy(bounce, acc_sh.at[pl.ds(start + t * BCH, BCH)])
            return carry

        lax.fori_loop(0, nch, zstep, 0)

        plsc.subcore_barrier()

        def group(j, carry):
            pltpu.sync_copy(src_hbm.at[wid, pl.ds(j * GRP, GRP)], srcv)
            pltpu.sync_copy(dst_hbm.at[wid, pl.ds(j * GRP, GRP)], dstv)

            def step(k, carry2):
                pltpu.async_copy(x_hbm.at[srcv.at[k]], rows, sem).wait()
                pltpu.sync_copy(rows, acc_sh.at[dstv.at[k]], add=True)
                return carry2

            return lax.fori_loop(0, GRP, step, carry)

        lax.fori_loop(0, NGRP, group, 0)

        plsc.subcore_barrier()

        def dstep(t, carry):
            off = start + t * BCH
            pltpu.sync_copy(acc_sh.at[pl.ds(off, BCH)], bounce)
            pltpu.sync_copy(bounce, acc_out.at[c, pl.ds(off, BCH)])
            return carry

        lax.fori_loop(0, nch, dstep, 0)

    return body


_sc_agg_128 = _make_sc_agg(D_IN)


def _make_sc_deg():
    mesh = plsc.VectorSubcoreMesh(core_axis_name="c", subcore_axis_name="s")

    @functools.partial(
        pl.kernel, mesh=mesh,
        out_type=jax.ShapeDtypeStruct((NC, N_NODES, D_IN), jnp.float32),
        scratch_types=[
            pltpu.VMEM((GRP, CHUNK), jnp.int32),
            pltpu.VMEM((CHUNK, D_IN), jnp.float32),
            pltpu.VMEM((BCH, D_IN), jnp.float32),
            pltpu.VMEM_SHARED((N_NODES, D_IN), jnp.float32),
        ],
    )
    def body(dst_hbm, ones_hbm, zrow_hbm, deg_out,
             dstv, ones, bounce, deg_sh):
        c = lax.axis_index("c")
        s = lax.axis_index("s")
        start, nch = _tile_rows(s)
        wid = c * NS + s

        pltpu.sync_copy(zrow_hbm, bounce)
        pltpu.sync_copy(ones_hbm, ones)

        def zstep(t, carry):
            pltpu.sync_copy(bounce, deg_sh.at[pl.ds(start + t * BCH, BCH)])
            return carry

        lax.fori_loop(0, nch, zstep, 0)

        plsc.subcore_barrier()

        def group(j, carry):
            pltpu.sync_copy(dst_hbm.at[wid, pl.ds(j * GRP, GRP)], dstv)

            def step(k, carry2):
                pltpu.sync_copy(ones, deg_sh.at[dstv.at[k]], add=True)
                return carry2

            return lax.fori_loop(0, GRP, step, carry)

        lax.fori_loop(0, NGRP, group, 0)

        plsc.subcore_barrier()

        def dstep(t, carry):
            off = start + t * BCH
            pltpu.sync_copy(deg_sh.at[pl.ds(off, BCH)], bounce)
            pltpu.sync_copy(bounce, deg_out.at[c, pl.ds(off, BCH)])
            return carry

        lax.fori_loop(0, nch, dstep, 0)

    return body


_sc_deg = _make_sc_deg()


def _dot_t(a, w):
    return lax.dot_general(a, w, (((1,), (1,)), ((), ())),
                           preferred_element_type=jnp.float32)


def _l2n(v):
    n = jnp.sqrt(jnp.sum(v * v, axis=1, keepdims=True))
    return v / jnp.maximum(n, 1e-12)


def _tc1_body(x_ref, acc_ref, deg_ref, wl_ref, bl_ref, wr_ref, h_ref):
    a = acc_ref[0] + acc_ref[1]
    dg = deg_ref[0, :, 0:1] + deg_ref[1, :, 0:1]
    agg = a / jnp.maximum(dg, 1.0)
    out = _dot_t(agg, wl_ref[...]) + bl_ref[...] + _dot_t(x_ref[...], wr_ref[...])
    h_ref[...] = jnp.maximum(_l2n(out), 0.0)


def _tc2_body(h_ref, acc_ref, deg_ref, batch_ref, wl_ref, bl_ref, wr_ref,
              wlin_ref, blin_ref, out_ref, gsum, gcnt):
    i = pl.program_id(0)

    @pl.when(i == 0)
    def _():
        gsum[...] = jnp.zeros_like(gsum)
        gcnt[...] = jnp.zeros_like(gcnt)

    a = acc_ref[0] + acc_ref[1]
    dg = deg_ref[0, :, 0:1] + deg_ref[1, :, 0:1]
    agg = a / jnp.maximum(dg, 1.0)
    out = _dot_t(agg, wl_ref[...]) + bl_ref[...] + _dot_t(h_ref[...], wr_ref[...])
    h2 = _l2n(out)

    b = batch_ref[0, 0]
    gid = lax.broadcasted_iota(jnp.int32, (NUM_GRAPHS, BLK), 0)
    mask = (b[None, :] == gid).astype(jnp.float32)
    gsum[...] += jnp.dot(mask, h2, preferred_element_type=jnp.float32)
    gcnt[...] += jnp.sum(mask, axis=1, keepdims=True)

    @pl.when(i == NB - 1)
    def _():
        g = gsum[...] / jnp.maximum(gcnt[...], 1.0)
        logits = _dot_t(wlin_ref[...], g) + blin_ref[0, 0]
        out_ref[...] = jax.nn.sigmoid(logits)


_tc1 = pl.pallas_call(
    _tc1_body,
    grid=(NB,),
    in_specs=[
        pl.BlockSpec((BLK, D_IN), lambda i: (i, 0)),
        pl.BlockSpec((NC, BLK, D_IN), lambda i: (0, i, 0)),
        pl.BlockSpec((NC, BLK, D_IN), lambda i: (0, i, 0)),
        pl.BlockSpec((HID, D_IN), lambda i: (0, 0)),
        pl.BlockSpec((1, HID), lambda i: (0, 0)),
        pl.BlockSpec((HID, D_IN), lambda i: (0, 0)),
    ],
    out_specs=pl.BlockSpec((BLK, HID), lambda i: (i, 0)),
    out_shape=jax.ShapeDtypeStruct((N_NODES, HID), jnp.float32),
)

_tc2 = pl.pallas_call(
    _tc2_body,
    grid=(NB,),
    in_specs=[
        pl.BlockSpec((BLK, HID), lambda i: (i, 0)),
        pl.BlockSpec((NC, BLK, HID), lambda i: (0, i, 0)),
        pl.BlockSpec((NC, BLK, D_IN), lambda i: (0, i, 0)),
        pl.BlockSpec((1, 1, BLK), lambda i: (i, 0, 0)),
        pl.BlockSpec((2 * HID, HID), lambda i: (0, 0)),
        pl.BlockSpec((1, 2 * HID), lambda i: (0, 0)),
        pl.BlockSpec((2 * HID, HID), lambda i: (0, 0)),
        pl.BlockSpec((1, 2 * HID), lambda i: (0, 0)),
        pl.BlockSpec((1, 1), lambda i: (0, 0)),
    ],
    out_specs=pl.BlockSpec((1, NUM_GRAPHS), lambda i: (0, 0)),
    out_shape=jax.ShapeDtypeStruct((1, NUM_GRAPHS), jnp.float32),
    scratch_shapes=[
        pltpu.VMEM((NUM_GRAPHS, 2 * HID), jnp.float32),
        pltpu.VMEM((NUM_GRAPHS, 1), jnp.float32),
    ],
)


def kernel(x, edge_index, batch, Wl1, bl1, Wr1, Wl2, bl2, Wr2, Wlin, blin):
    src = edge_index[0].astype(jnp.int32).reshape(NW, KCH, CHUNK)
    dst = edge_index[1].astype(jnp.int32).reshape(NW, KCH, CHUNK)
    batch3 = batch.astype(jnp.int32).reshape(NB, 1, BLK)
    zrow = jnp.zeros((BCH, D_IN), jnp.float32)
    ones = jnp.ones((CHUNK, D_IN), jnp.float32)

    deg = _sc_deg(dst, ones, zrow)
    acc1 = _sc_agg_128(x, src, dst, zrow)
    h = _tc1(x, acc1, deg, Wl1, bl1.reshape(1, HID), Wr1)
    acc2 = _sc_agg_128(h, src, dst, zrow)
    out = _tc2(h, acc2, deg, batch3, Wl2, bl2.reshape(1, 2 * HID), Wr2,
               Wlin, blin.reshape(1, 1))
    return out.reshape(NUM_GRAPHS, 1)


# speedup vs baseline: 7.6577x; 1.0179x over previous
import functools

import jax
import jax.numpy as jnp
from jax import lax
from jax.experimental import pallas as pl
from jax.experimental.pallas import tpu as pltpu
from jax.experimental.pallas import tpu_sc as plsc

N_NODES = 10000
N_EDGES = 320000
D_IN = 128
HID = 128

NC = 2
NS = 16
NW = NC * NS

CHUNK = 125
EPW = N_EDGES // NW
KCH = EPW // CHUNK
GRP = 8
NGRP = KCH // GRP
ROWS_T = 624
ROWS_LAST = N_NODES - 15 * ROWS_T
BCH = 16

NB = 5
BLK = N_NODES // NB
NUM_GRAPHS = 64


def _per_tile_zero(src_hbm, dst_sh, s):
    start = s * ROWS_T

    @pl.when(s < NS - 1)
    def _():
        pltpu.sync_copy(src_hbm.at[pl.ds(0, ROWS_T)],
                        dst_sh.at[pl.ds(start, ROWS_T)])

    @pl.when(s == NS - 1)
    def _():
        pltpu.sync_copy(src_hbm, dst_sh.at[pl.ds(start, ROWS_LAST)])


def _per_tile_drain(src_sh, dst_hbm, c, s):
    start = s * ROWS_T

    @pl.when(s < NS - 1)
    def _():
        pltpu.sync_copy(src_sh.at[pl.ds(start, ROWS_T)],
                        dst_hbm.at[c, pl.ds(start, ROWS_T)])

    @pl.when(s == NS - 1)
    def _():
        pltpu.sync_copy(src_sh.at[pl.ds(start, ROWS_LAST)],
                        dst_hbm.at[c, pl.ds(start, ROWS_LAST)])


def _make_sc_agg(feat):
    mesh = plsc.VectorSubcoreMesh(core_axis_name="c", subcore_axis_name="s")

    @functools.partial(
        pl.kernel, mesh=mesh,
        out_type=jax.ShapeDtypeStruct((NC, N_NODES, feat), jnp.float32),
        scratch_types=[
            pltpu.VMEM((GRP, CHUNK), jnp.int32),
            pltpu.VMEM((GRP, CHUNK), jnp.int32),
            pltpu.VMEM((CHUNK, feat), jnp.float32),
            pltpu.VMEM_SHARED((N_NODES, feat), jnp.float32),
            pltpu.SemaphoreType.DMA,
        ],
    )
    def body(x_hbm, src_hbm, dst_hbm, zrow_hbm, acc_out,
             srcv, dstv, rows, acc_sh, sem):
        c = lax.axis_index("c")
        s = lax.axis_index("s")
        wid = c * NS + s

        _per_tile_zero(zrow_hbm, acc_sh, s)

        plsc.subcore_barrier()

        def group(j, carry):
            pltpu.sync_copy(src_hbm.at[wid, pl.ds(j * GRP, GRP)], srcv)
            pltpu.sync_copy(dst_hbm.at[wid, pl.ds(j * GRP, GRP)], dstv)

            def step(k, carry2):
                pltpu.async_copy(x_hbm.at[srcv.at[k]], rows, sem).wait()
                pltpu.sync_copy(rows, acc_sh.at[dstv.at[k]], add=True)
                return carry2

            return lax.fori_loop(0, GRP, step, carry)

        lax.fori_loop(0, NGRP, group, 0)

        plsc.subcore_barrier()

        _per_tile_drain(acc_sh, acc_out, c, s)

    return body


_sc_agg_128 = _make_sc_agg(D_IN)


def _make_sc_deg():
    mesh = plsc.VectorSubcoreMesh(core_axis_name="c", subcore_axis_name="s")

    @functools.partial(
        pl.kernel, mesh=mesh,
        out_type=jax.ShapeDtypeStruct((NC, N_NODES, D_IN), jnp.float32),
        scratch_types=[
            pltpu.VMEM((GRP, CHUNK), jnp.int32),
            pltpu.VMEM((CHUNK, D_IN), jnp.float32),
            pltpu.VMEM_SHARED((N_NODES, D_IN), jnp.float32),
        ],
    )
    def body(dst_hbm, ones_hbm, zrow_hbm, deg_out, dstv, ones, deg_sh):
        c = lax.axis_index("c")
        s = lax.axis_index("s")
        wid = c * NS + s

        pltpu.sync_copy(ones_hbm, ones)
        _per_tile_zero(zrow_hbm, deg_sh, s)

        plsc.subcore_barrier()

        def group(j, carry):
            pltpu.sync_copy(dst_hbm.at[wid, pl.ds(j * GRP, GRP)], dstv)

            def step(k, carry2):
                pltpu.sync_copy(ones, deg_sh.at[dstv.at[k]], add=True)
                return carry2

            return lax.fori_loop(0, GRP, step, carry)

        lax.fori_loop(0, NGRP, group, 0)

        plsc.subcore_barrier()

        _per_tile_drain(deg_sh, deg_out, c, s)

    return body


_sc_deg = _make_sc_deg()


def _dot_t(a, w):
    return lax.dot_general(a, w, (((1,), (1,)), ((), ())),
                           preferred_element_type=jnp.float32)


def _l2n(v):
    n = jnp.sqrt(jnp.sum(v * v, axis=1, keepdims=True))
    return v / jnp.maximum(n, 1e-12)


def _tc1_body(x_ref, acc_ref, deg_ref, wl_ref, bl_ref, wr_ref, h_ref):
    a = acc_ref[0] + acc_ref[1]
    dg = deg_ref[0, :, 0:1] + deg_ref[1, :, 0:1]
    agg = a / jnp.maximum(dg, 1.0)
    out = _dot_t(agg, wl_ref[...]) + bl_ref[...] + _dot_t(x_ref[...], wr_ref[...])
    h_ref[...] = jnp.maximum(_l2n(out), 0.0)


def _tc2_body(h_ref, acc_ref, deg_ref, batch_ref, wl_ref, bl_ref, wr_ref,
              wlin_ref, blin_ref, out_ref, gsum, gcnt):
    i = pl.program_id(0)

    @pl.when(i == 0)
    def _():
        gsum[...] = jnp.zeros_like(gsum)
        gcnt[...] = jnp.zeros_like(gcnt)

    a = acc_ref[0] + acc_ref[1]
    dg = deg_ref[0, :, 0:1] + deg_ref[1, :, 0:1]
    agg = a / jnp.maximum(dg, 1.0)
    out = _dot_t(agg, wl_ref[...]) + bl_ref[...] + _dot_t(h_ref[...], wr_ref[...])
    h2 = _l2n(out)

    b = batch_ref[0, 0]
    gid = lax.broadcasted_iota(jnp.int32, (NUM_GRAPHS, BLK), 0)
    mask = (b[None, :] == gid).astype(jnp.float32)
    gsum[...] += jnp.dot(mask, h2, preferred_element_type=jnp.float32)
    gcnt[...] += jnp.sum(mask, axis=1, keepdims=True)

    @pl.when(i == NB - 1)
    def _():
        g = gsum[...] / jnp.maximum(gcnt[...], 1.0)
        logits = _dot_t(wlin_ref[...], g) + blin_ref[0, 0]
        out_ref[...] = jax.nn.sigmoid(logits)


_tc1 = pl.pallas_call(
    _tc1_body,
    grid=(NB,),
    in_specs=[
        pl.BlockSpec((BLK, D_IN), lambda i: (i, 0)),
        pl.BlockSpec((NC, BLK, D_IN), lambda i: (0, i, 0)),
        pl.BlockSpec((NC, BLK, D_IN), lambda i: (0, i, 0)),
        pl.BlockSpec((HID, D_IN), lambda i: (0, 0)),
        pl.BlockSpec((1, HID), lambda i: (0, 0)),
        pl.BlockSpec((HID, D_IN), lambda i: (0, 0)),
    ],
    out_specs=pl.BlockSpec((BLK, HID), lambda i: (i, 0)),
    out_shape=jax.ShapeDtypeStruct((N_NODES, HID), jnp.float32),
)

_tc2 = pl.pallas_call(
    _tc2_body,
    grid=(NB,),
    in_specs=[
        pl.BlockSpec((BLK, HID), lambda i: (i, 0)),
        pl.BlockSpec((NC, BLK, HID), lambda i: (0, i, 0)),
        pl.BlockSpec((NC, BLK, D_IN), lambda i: (0, i, 0)),
        pl.BlockSpec((1, 1, BLK), lambda i: (i, 0, 0)),
        pl.BlockSpec((2 * HID, HID), lambda i: (0, 0)),
        pl.BlockSpec((1, 2 * HID), lambda i: (0, 0)),
        pl.BlockSpec((2 * HID, HID), lambda i: (0, 0)),
        pl.BlockSpec((1, 2 * HID), lambda i: (0, 0)),
        pl.BlockSpec((1, 1), lambda i: (0, 0)),
    ],
    out_specs=pl.BlockSpec((1, NUM_GRAPHS), lambda i: (0, 0)),
    out_shape=jax.ShapeDtypeStruct((1, NUM_GRAPHS), jnp.float32),
    scratch_shapes=[
        pltpu.VMEM((NUM_GRAPHS, 2 * HID), jnp.float32),
        pltpu.VMEM((NUM_GRAPHS, 1), jnp.float32),
    ],
)


def kernel(x, edge_index, batch, Wl1, bl1, Wr1, Wl2, bl2, Wr2, Wlin, blin):
    src = edge_index[0].astype(jnp.int32).reshape(NW, KCH, CHUNK)
    dst = edge_index[1].astype(jnp.int32).reshape(NW, KCH, CHUNK)
    batch3 = batch.astype(jnp.int32).reshape(NB, 1, BLK)
    zrow = jnp.zeros((ROWS_LAST, D_IN), jnp.float32)
    ones = jnp.ones((CHUNK, D_IN), jnp.float32)

    deg = _sc_deg(dst, ones, zrow)
    acc1 = _sc_agg_128(x, src, dst, zrow)
    h = _tc1(x, acc1, deg, Wl1, bl1.reshape(1, HID), Wr1)
    acc2 = _sc_agg_128(h, src, dst, zrow)
    out = _tc2(h, acc2, deg, batch3, Wl2, bl2.reshape(1, 2 * HID), Wr2,
               Wlin, blin.reshape(1, 1))
    return out.reshape(NUM_GRAPHS, 1)


# speedup vs baseline: 9.7304x; 1.2707x over previous
import functools

import jax
import jax.numpy as jnp
from jax import lax
from jax.experimental import pallas as pl
from jax.experimental.pallas import tpu as pltpu
from jax.experimental.pallas import tpu_sc as plsc

N_NODES = 10000
N_EDGES = 320000
D_IN = 128
HID = 128

NC = 2
NS = 16
NW = NC * NS

CHUNK = 125
EPW = N_EDGES // NW
KCH = EPW // CHUNK
GRP = 8
NGRP = KCH // GRP
ROWS_T = 624
ROWS_LAST = N_NODES - 15 * ROWS_T
BCH = 16

NB = 5
BLK = N_NODES // NB
NUM_GRAPHS = 64


def _per_tile_zero(src_hbm, dst_sh, s):
    start = s * ROWS_T

    @pl.when(s < NS - 1)
    def _():
        pltpu.sync_copy(src_hbm.at[pl.ds(0, ROWS_T)],
                        dst_sh.at[pl.ds(start, ROWS_T)])

    @pl.when(s == NS - 1)
    def _():
        pltpu.sync_copy(src_hbm, dst_sh.at[pl.ds(start, ROWS_LAST)])


def _per_tile_drain(src_sh, dst_hbm, c, s):
    start = s * ROWS_T

    @pl.when(s < NS - 1)
    def _():
        pltpu.sync_copy(src_sh.at[pl.ds(start, ROWS_T)],
                        dst_hbm.at[c, pl.ds(start, ROWS_T)])

    @pl.when(s == NS - 1)
    def _():
        pltpu.sync_copy(src_sh.at[pl.ds(start, ROWS_LAST)],
                        dst_hbm.at[c, pl.ds(start, ROWS_LAST)])


def _make_sc_agg(feat):
    mesh = plsc.VectorSubcoreMesh(core_axis_name="c", subcore_axis_name="s")

    @functools.partial(
        pl.kernel, mesh=mesh,
        out_type=jax.ShapeDtypeStruct((NC, N_NODES, feat), jnp.float32),
        scratch_types=[
            pltpu.VMEM((2, GRP, CHUNK), jnp.int32),
            pltpu.VMEM((2, GRP, CHUNK), jnp.int32),
            pltpu.VMEM((2, CHUNK, feat), jnp.float32),
            pltpu.VMEM_SHARED((N_NODES, feat), jnp.float32),
            pltpu.SemaphoreType.DMA((2,)),
            pltpu.SemaphoreType.DMA((2,)),
            pltpu.SemaphoreType.DMA,
        ],
    )
    def body(x_hbm, src_hbm, dst_hbm, zrow_hbm, acc_out,
             srcv, dstv, rows, acc_sh, gsem, ssem, isem):
        c = lax.axis_index("c")
        s = lax.axis_index("s")
        wid = c * NS + s

        _per_tile_zero(zrow_hbm, acc_sh, s)

        plsc.subcore_barrier()

        def stage(g, b):
            return (
                pltpu.make_async_copy(src_hbm.at[wid, pl.ds(g * GRP, GRP)],
                                      srcv.at[b], isem),
                pltpu.make_async_copy(dst_hbm.at[wid, pl.ds(g * GRP, GRP)],
                                      dstv.at[b], isem),
            )

        def gath(k, p):
            g = k // GRP
            return pltpu.make_async_copy(
                x_hbm.at[srcv.at[g % 2, k % GRP]], rows.at[p], gsem.at[p])

        def scat(k, p):
            g = k // GRP
            return pltpu.make_async_copy(
                rows.at[p], acc_sh.at[dstv.at[g % 2, k % GRP]], ssem.at[p])

        for d in stage(0, 0):
            d.start()
            d.wait()
        gath(0, 0).start()

        def step(k, carry):
            p = lax.rem(k, 2)
            q = 1 - p
            g = k // GRP
            r = lax.rem(k, GRP)

            @pl.when(jnp.logical_and(r == 0, g < NGRP - 1))
            def _():
                for d in stage(g + 1, (g + 1) % 2):
                    d.start()

            gath(k, p).wait()
            scat(k, p).start(add=True)

            @pl.when(k >= 1)
            def _():
                scat(k - 1, q).wait()

            @pl.when(k < KCH - 1)
            def _():
                @pl.when(r == GRP - 1)
                def _():
                    for d in stage(g + 1, (g + 1) % 2):
                        d.wait()

                gath(k + 1, q).start()

            return carry

        lax.fori_loop(0, KCH, step, 0)
        scat(KCH - 1, (KCH - 1) % 2).wait()

        plsc.subcore_barrier()

        _per_tile_drain(acc_sh, acc_out, c, s)

    return body


_sc_agg_128 = _make_sc_agg(D_IN)


def _make_sc_deg():
    mesh = plsc.VectorSubcoreMesh(core_axis_name="c", subcore_axis_name="s")

    @functools.partial(
        pl.kernel, mesh=mesh,
        out_type=jax.ShapeDtypeStruct((NC, N_NODES, D_IN), jnp.float32),
        scratch_types=[
            pltpu.VMEM((GRP, CHUNK), jnp.int32),
            pltpu.VMEM((CHUNK, D_IN), jnp.float32),
            pltpu.VMEM_SHARED((N_NODES, D_IN), jnp.float32),
        ],
    )
    def body(dst_hbm, ones_hbm, zrow_hbm, deg_out, dstv, ones, deg_sh):
        c = lax.axis_index("c")
        s = lax.axis_index("s")
        wid = c * NS + s

        pltpu.sync_copy(ones_hbm, ones)
        _per_tile_zero(zrow_hbm, deg_sh, s)

        plsc.subcore_barrier()

        def group(j, carry):
            pltpu.sync_copy(dst_hbm.at[wid, pl.ds(j * GRP, GRP)], dstv)

            def step(k, carry2):
                pltpu.sync_copy(ones, deg_sh.at[dstv.at[k]], add=True)
                return carry2

            return lax.fori_loop(0, GRP, step, carry)

        lax.fori_loop(0, NGRP, group, 0)

        plsc.subcore_barrier()

        _per_tile_drain(deg_sh, deg_out, c, s)

    return body


_sc_deg = _make_sc_deg()


def _dot_t(a, w):
    return lax.dot_general(a, w, (((1,), (1,)), ((), ())),
                           preferred_element_type=jnp.float32)


def _l2n(v):
    n = jnp.sqrt(jnp.sum(v * v, axis=1, keepdims=True))
    return v / jnp.maximum(n, 1e-12)


def _tc1_body(x_ref, acc_ref, deg_ref, wl_ref, bl_ref, wr_ref, h_ref):
    a = acc_ref[0] + acc_ref[1]
    dg = deg_ref[0, :, 0:1] + deg_ref[1, :, 0:1]
    agg = a / jnp.maximum(dg, 1.0)
    out = _dot_t(agg, wl_ref[...]) + bl_ref[...] + _dot_t(x_ref[...], wr_ref[...])
    h_ref[...] = jnp.maximum(_l2n(out), 0.0)


def _tc2_body(h_ref, acc_ref, deg_ref, batch_ref, wl_ref, bl_ref, wr_ref,
              wlin_ref, blin_ref, out_ref, gsum, gcnt):
    i = pl.program_id(0)

    @pl.when(i == 0)
    def _():
        gsum[...] = jnp.zeros_like(gsum)
        gcnt[...] = jnp.zeros_like(gcnt)

    a = acc_ref[0] + acc_ref[1]
    dg = deg_ref[0, :, 0:1] + deg_ref[1, :, 0:1]
    agg = a / jnp.maximum(dg, 1.0)
    out = _dot_t(agg, wl_ref[...]) + bl_ref[...] + _dot_t(h_ref[...], wr_ref[...])
    h2 = _l2n(out)

    b = batch_ref[0, 0]
    gid = lax.broadcasted_iota(jnp.int32, (NUM_GRAPHS, BLK), 0)
    mask = (b[None, :] == gid).astype(jnp.float32)
    gsum[...] += jnp.dot(mask, h2, preferred_element_type=jnp.float32)
    gcnt[...] += jnp.sum(mask, axis=1, keepdims=True)

    @pl.when(i == NB - 1)
    def _():
        g = gsum[...] / jnp.maximum(gcnt[...], 1.0)
        logits = _dot_t(wlin_ref[...], g) + blin_ref[0, 0]
        out_ref[...] = jax.nn.sigmoid(logits)


_tc1 = pl.pallas_call(
    _tc1_body,
    grid=(NB,),
    in_specs=[
        pl.BlockSpec((BLK, D_IN), lambda i: (i, 0)),
        pl.BlockSpec((NC, BLK, D_IN), lambda i: (0, i, 0)),
        pl.BlockSpec((NC, BLK, D_IN), lambda i: (0, i, 0)),
        pl.BlockSpec((HID, D_IN), lambda i: (0, 0)),
        pl.BlockSpec((1, HID), lambda i: (0, 0)),
        pl.BlockSpec((HID, D_IN), lambda i: (0, 0)),
    ],
    out_specs=pl.BlockSpec((BLK, HID), lambda i: (i, 0)),
    out_shape=jax.ShapeDtypeStruct((N_NODES, HID), jnp.float32),
)

_tc2 = pl.pallas_call(
    _tc2_body,
    grid=(NB,),
    in_specs=[
        pl.BlockSpec((BLK, HID), lambda i: (i, 0)),
        pl.BlockSpec((NC, BLK, HID), lambda i: (0, i, 0)),
        pl.BlockSpec((NC, BLK, D_IN), lambda i: (0, i, 0)),
        pl.BlockSpec((1, 1, BLK), lambda i: (i, 0, 0)),
        pl.BlockSpec((2 * HID, HID), lambda i: (0, 0)),
        pl.BlockSpec((1, 2 * HID), lambda i: (0, 0)),
        pl.BlockSpec((2 * HID, HID), lambda i: (0, 0)),
        pl.BlockSpec((1, 2 * HID), lambda i: (0, 0)),
        pl.BlockSpec((1, 1), lambda i: (0, 0)),
    ],
    out_specs=pl.BlockSpec((1, NUM_GRAPHS), lambda i: (0, 0)),
    out_shape=jax.ShapeDtypeStruct((1, NUM_GRAPHS), jnp.float32),
    scratch_shapes=[
        pltpu.VMEM((NUM_GRAPHS, 2 * HID), jnp.float32),
        pltpu.VMEM((NUM_GRAPHS, 1), jnp.float32),
    ],
)


def kernel(x, edge_index, batch, Wl1, bl1, Wr1, Wl2, bl2, Wr2, Wlin, blin):
    src = edge_index[0].astype(jnp.int32).reshape(NW, KCH, CHUNK)
    dst = edge_index[1].astype(jnp.int32).reshape(NW, KCH, CHUNK)
    batch3 = batch.astype(jnp.int32).reshape(NB, 1, BLK)
    zrow = jnp.zeros((ROWS_LAST, D_IN), jnp.float32)
    ones = jnp.ones((CHUNK, D_IN), jnp.float32)

    deg = _sc_deg(dst, ones, zrow)
    acc1 = _sc_agg_128(x, src, dst, zrow)
    h = _tc1(x, acc1, deg, Wl1, bl1.reshape(1, HID), Wr1)
    acc2 = _sc_agg_128(h, src, dst, zrow)
    out = _tc2(h, acc2, deg, batch3, Wl2, bl2.reshape(1, 2 * HID), Wr2,
               Wlin, blin.reshape(1, 1))
    return out.reshape(NUM_GRAPHS, 1)


# speedup vs baseline: 9.8778x; 1.0151x over previous
import functools

import jax
import jax.numpy as jnp
from jax import lax
from jax.experimental import pallas as pl
from jax.experimental.pallas import tpu as pltpu
from jax.experimental.pallas import tpu_sc as plsc

N_NODES = 10000
N_EDGES = 320000
D_IN = 128
HID = 128

NC = 2
NS = 16
NW = NC * NS

CHUNK = 125
EPW = N_EDGES // NW
KCH = EPW // CHUNK
GRP = 8
NGRP = KCH // GRP
ROWS_T = 624
ROWS_LAST = N_NODES - 15 * ROWS_T
BCH = 16

NB = 5
BLK = N_NODES // NB
NUM_GRAPHS = 64


def _per_tile_zero(src_hbm, dst_sh, s):
    start = s * ROWS_T

    @pl.when(s < NS - 1)
    def _():
        pltpu.sync_copy(src_hbm.at[pl.ds(0, ROWS_T)],
                        dst_sh.at[pl.ds(start, ROWS_T)])

    @pl.when(s == NS - 1)
    def _():
        pltpu.sync_copy(src_hbm, dst_sh.at[pl.ds(start, ROWS_LAST)])


def _per_tile_drain(src_sh, dst_hbm, c, s):
    start = s * ROWS_T

    @pl.when(s < NS - 1)
    def _():
        pltpu.sync_copy(src_sh.at[pl.ds(start, ROWS_T)],
                        dst_hbm.at[c, pl.ds(start, ROWS_T)])

    @pl.when(s == NS - 1)
    def _():
        pltpu.sync_copy(src_sh.at[pl.ds(start, ROWS_LAST)],
                        dst_hbm.at[c, pl.ds(start, ROWS_LAST)])


def _make_sc_agg(feat):
    mesh = plsc.VectorSubcoreMesh(core_axis_name="c", subcore_axis_name="s")

    @functools.partial(
        pl.kernel, mesh=mesh,
        out_type=jax.ShapeDtypeStruct((NC, N_NODES, feat), jnp.float32),
        scratch_types=[
            pltpu.VMEM((2, GRP, CHUNK), jnp.int32),
            pltpu.VMEM((2, GRP, CHUNK), jnp.int32),
            pltpu.VMEM((2, CHUNK, feat), jnp.float32),
            pltpu.VMEM_SHARED((N_NODES, feat), jnp.float32),
            pltpu.SemaphoreType.DMA((2,)),
            pltpu.SemaphoreType.DMA((2,)),
            pltpu.SemaphoreType.DMA,
        ],
    )
    def body(x_hbm, src_hbm, dst_hbm, zrow_hbm, acc_out,
             srcv, dstv, rows, acc_sh, gsem, ssem, isem):
        c = lax.axis_index("c")
        s = lax.axis_index("s")
        wid = c * NS + s

        _per_tile_zero(zrow_hbm, acc_sh, s)

        plsc.subcore_barrier()

        def stage(g, b):
            return (
                pltpu.make_async_copy(src_hbm.at[wid, pl.ds(g * GRP, GRP)],
                                      srcv.at[b], isem),
                pltpu.make_async_copy(dst_hbm.at[wid, pl.ds(g * GRP, GRP)],
                                      dstv.at[b], isem),
            )

        def gath(k, p):
            g = k // GRP
            return pltpu.make_async_copy(
                x_hbm.at[srcv.at[g % 2, k % GRP]], rows.at[p], gsem.at[p])

        def scat(k, p):
            g = k // GRP
            return pltpu.make_async_copy(
                rows.at[p], acc_sh.at[dstv.at[g % 2, k % GRP]], ssem.at[p])

        for d in stage(0, 0):
            d.start()
            d.wait()
        gath(0, 0).start()

        def step(k, carry):
            p = lax.rem(k, 2)
            q = 1 - p
            g = k // GRP
            r = lax.rem(k, GRP)

            @pl.when(k >= 1)
            def _():
                scat(k - 1, q).wait()

            @pl.when(jnp.logical_and(r == 0, g < NGRP - 1))
            def _():
                for d in stage(g + 1, (g + 1) % 2):
                    d.start()

            gath(k, p).wait()
            scat(k, p).start(add=True)

            @pl.when(k < KCH - 1)
            def _():
                @pl.when(r == GRP - 1)
                def _():
                    for d in stage(g + 1, (g + 1) % 2):
                        d.wait()

                gath(k + 1, q).start()

            return carry

        lax.fori_loop(0, KCH, step, 0)
        scat(KCH - 1, (KCH - 1) % 2).wait()

        plsc.subcore_barrier()

        _per_tile_drain(acc_sh, acc_out, c, s)

    return body


_sc_agg_128 = _make_sc_agg(D_IN)


def _make_sc_deg():
    mesh = plsc.VectorSubcoreMesh(core_axis_name="c", subcore_axis_name="s")

    @functools.partial(
        pl.kernel, mesh=mesh,
        out_type=jax.ShapeDtypeStruct((NC, N_NODES, D_IN), jnp.float32),
        scratch_types=[
            pltpu.VMEM((2, GRP, CHUNK), jnp.int32),
            pltpu.VMEM((CHUNK, D_IN), jnp.float32),
            pltpu.VMEM_SHARED((N_NODES, D_IN), jnp.float32),
            pltpu.SemaphoreType.DMA((2,)),
            pltpu.SemaphoreType.DMA,
        ],
    )
    def body(dst_hbm, ones_hbm, zrow_hbm, deg_out, dstv, ones, deg_sh,
             ssem, isem):
        c = lax.axis_index("c")
        s = lax.axis_index("s")
        wid = c * NS + s

        pltpu.sync_copy(ones_hbm, ones)
        _per_tile_zero(zrow_hbm, deg_sh, s)

        plsc.subcore_barrier()

        def stage(g):
            return pltpu.make_async_copy(
                dst_hbm.at[wid, pl.ds(g * GRP, GRP)], dstv.at[g % 2], isem)

        def scat(k, p):
            g = k // GRP
            return pltpu.make_async_copy(
                ones, deg_sh.at[dstv.at[g % 2, k % GRP]], ssem.at[p])

        d = stage(0)
        d.start()
        d.wait()

        def step(k, carry):
            p = lax.rem(k, 2)
            g = k // GRP
            r = lax.rem(k, GRP)

            @pl.when(k >= 1)
            def _():
                scat(k - 1, 1 - p).wait()

            @pl.when(jnp.logical_and(r == 0, g < NGRP - 1))
            def _():
                stage(g + 1).start()

            @pl.when(jnp.logical_and(r == GRP - 1, g < NGRP - 1))
            def _():
                stage(g + 1).wait()

            scat(k, p).start(add=True)

            return carry

        lax.fori_loop(0, KCH, step, 0)
        scat(KCH - 1, (KCH - 1) % 2).wait()

        plsc.subcore_barrier()

        _per_tile_drain(deg_sh, deg_out, c, s)

    return body


_sc_deg = _make_sc_deg()


def _dot_t(a, w):
    return lax.dot_general(a, w, (((1,), (1,)), ((), ())),
                           preferred_element_type=jnp.float32)


def _l2n(v):
    n = jnp.sqrt(jnp.sum(v * v, axis=1, keepdims=True))
    return v / jnp.maximum(n, 1e-12)


def _tc1_body(x_ref, acc_ref, deg_ref, wl_ref, bl_ref, wr_ref, h_ref):
    a = acc_ref[0] + acc_ref[1]
    dg = deg_ref[0, :, 0:1] + deg_ref[1, :, 0:1]
    agg = a / jnp.maximum(dg, 1.0)
    out = _dot_t(agg, wl_ref[...]) + bl_ref[...] + _dot_t(x_ref[...], wr_ref[...])
    h_ref[...] = jnp.maximum(_l2n(out), 0.0)


def _tc2_body(h_ref, acc_ref, deg_ref, batch_ref, wl_ref, bl_ref, wr_ref,
              wlin_ref, blin_ref, out_ref, gsum, gcnt):
    i = pl.program_id(0)

    @pl.when(i == 0)
    def _():
        gsum[...] = jnp.zeros_like(gsum)
        gcnt[...] = jnp.zeros_like(gcnt)

    a = acc_ref[0] + acc_ref[1]
    dg = deg_ref[0, :, 0:1] + deg_ref[1, :, 0:1]
    agg = a / jnp.maximum(dg, 1.0)
    out = _dot_t(agg, wl_ref[...]) + bl_ref[...] + _dot_t(h_ref[...], wr_ref[...])
    h2 = _l2n(out)

    b = batch_ref[0, 0]
    gid = lax.broadcasted_iota(jnp.int32, (NUM_GRAPHS, BLK), 0)
    mask = (b[None, :] == gid).astype(jnp.float32)
    gsum[...] += jnp.dot(mask, h2, preferred_element_type=jnp.float32)
    gcnt[...] += jnp.sum(mask, axis=1, keepdims=True)

    @pl.when(i == NB - 1)
    def _():
        g = gsum[...] / jnp.maximum(gcnt[...], 1.0)
        logits = _dot_t(wlin_ref[...], g) + blin_ref[0, 0]
        out_ref[...] = jax.nn.sigmoid(logits)


_tc1 = pl.pallas_call(
    _tc1_body,
    grid=(NB,),
    in_specs=[
        pl.BlockSpec((BLK, D_IN), lambda i: (i, 0)),
        pl.BlockSpec((NC, BLK, D_IN), lambda i: (0, i, 0)),
        pl.BlockSpec((NC, BLK, D_IN), lambda i: (0, i, 0)),
        pl.BlockSpec((HID, D_IN), lambda i: (0, 0)),
        pl.BlockSpec((1, HID), lambda i: (0, 0)),
        pl.BlockSpec((HID, D_IN), lambda i: (0, 0)),
    ],
    out_specs=pl.BlockSpec((BLK, HID), lambda i: (i, 0)),
    out_shape=jax.ShapeDtypeStruct((N_NODES, HID), jnp.float32),
)

_tc2 = pl.pallas_call(
    _tc2_body,
    grid=(NB,),
    in_specs=[
        pl.BlockSpec((BLK, HID), lambda i: (i, 0)),
        pl.BlockSpec((NC, BLK, HID), lambda i: (0, i, 0)),
        pl.BlockSpec((NC, BLK, D_IN), lambda i: (0, i, 0)),
        pl.BlockSpec((1, 1, BLK), lambda i: (i, 0, 0)),
        pl.BlockSpec((2 * HID, HID), lambda i: (0, 0)),
        pl.BlockSpec((1, 2 * HID), lambda i: (0, 0)),
        pl.BlockSpec((2 * HID, HID), lambda i: (0, 0)),
        pl.BlockSpec((1, 2 * HID), lambda i: (0, 0)),
        pl.BlockSpec((1, 1), lambda i: (0, 0)),
    ],
    out_specs=pl.BlockSpec((1, NUM_GRAPHS), lambda i: (0, 0)),
    out_shape=jax.ShapeDtypeStruct((1, NUM_GRAPHS), jnp.float32),
    scratch_shapes=[
        pltpu.VMEM((NUM_GRAPHS, 2 * HID), jnp.float32),
        pltpu.VMEM((NUM_GRAPHS, 1), jnp.float32),
    ],
)


def kernel(x, edge_index, batch, Wl1, bl1, Wr1, Wl2, bl2, Wr2, Wlin, blin):
    src = edge_index[0].astype(jnp.int32).reshape(NW, KCH, CHUNK)
    dst = edge_index[1].astype(jnp.int32).reshape(NW, KCH, CHUNK)
    batch3 = batch.astype(jnp.int32).reshape(NB, 1, BLK)
    zrow = jnp.zeros((ROWS_LAST, D_IN), jnp.float32)
    ones = jnp.ones((CHUNK, D_IN), jnp.float32)

    deg = _sc_deg(dst, ones, zrow)
    acc1 = _sc_agg_128(x, src, dst, zrow)
    h = _tc1(x, acc1, deg, Wl1, bl1.reshape(1, HID), Wr1)
    acc2 = _sc_agg_128(h, src, dst, zrow)
    out = _tc2(h, acc2, deg, batch3, Wl2, bl2.reshape(1, 2 * HID), Wr2,
               Wlin, blin.reshape(1, 1))
    return out.reshape(NUM_GRAPHS, 1)


# speedup vs baseline: 11.0969x; 1.1234x over previous
import functools

import jax
import jax.numpy as jnp
from jax import lax
from jax.experimental import pallas as pl
from jax.experimental.pallas import tpu as pltpu
from jax.experimental.pallas import tpu_sc as plsc

N_NODES = 10000
N_EDGES = 320000
D_IN = 128
HID = 128

NC = 2
NS = 16
NW = NC * NS

CHUNK = 125
EPW = N_EDGES // NW
KCH = EPW // CHUNK
GRP = 8
NGRP = KCH // GRP
ROWS_T = 624
ROWS_LAST = N_NODES - 15 * ROWS_T
BCH = 16

NB = 5
BLK = N_NODES // NB
NUM_GRAPHS = 64


def _per_tile_zero(src_hbm, dst_sh, s):
    start = s * ROWS_T

    @pl.when(s < NS - 1)
    def _():
        pltpu.sync_copy(src_hbm.at[pl.ds(0, ROWS_T)],
                        dst_sh.at[pl.ds(start, ROWS_T)])

    @pl.when(s == NS - 1)
    def _():
        pltpu.sync_copy(src_hbm, dst_sh.at[pl.ds(start, ROWS_LAST)])


def _per_tile_drain(src_sh, dst_hbm, c, s):
    start = s * ROWS_T

    @pl.when(s < NS - 1)
    def _():
        pltpu.sync_copy(src_sh.at[pl.ds(start, ROWS_T)],
                        dst_hbm.at[c, pl.ds(start, ROWS_T)])

    @pl.when(s == NS - 1)
    def _():
        pltpu.sync_copy(src_sh.at[pl.ds(start, ROWS_LAST)],
                        dst_hbm.at[c, pl.ds(start, ROWS_LAST)])


def _make_sc_agg(feat):
    mesh = plsc.VectorSubcoreMesh(core_axis_name="c", subcore_axis_name="s")

    @functools.partial(
        pl.kernel, mesh=mesh,
        out_type=jax.ShapeDtypeStruct((NC, N_NODES, feat), jnp.float32),
        scratch_types=[
            pltpu.VMEM((2, GRP, CHUNK), jnp.int32),
            pltpu.VMEM((2, GRP, CHUNK), jnp.int32),
            pltpu.VMEM((2, CHUNK, feat), jnp.float32),
            pltpu.VMEM_SHARED((N_NODES, feat), jnp.float32),
            pltpu.SemaphoreType.DMA((2,)),
            pltpu.SemaphoreType.DMA((2,)),
            pltpu.SemaphoreType.DMA,
        ],
    )
    def body(x_hbm, src_hbm, dst_hbm, zrow_hbm, acc_out,
             srcv, dstv, rows, acc_sh, gsem, ssem, isem):
        c = lax.axis_index("c")
        s = lax.axis_index("s")
        wid = c * NS + s

        _per_tile_zero(zrow_hbm, acc_sh, s)

        plsc.subcore_barrier()

        def stage(g, b):
            return (
                pltpu.make_async_copy(src_hbm.at[wid, pl.ds(g * GRP, GRP)],
                                      srcv.at[b], isem),
                pltpu.make_async_copy(dst_hbm.at[wid, pl.ds(g * GRP, GRP)],
                                      dstv.at[b], isem),
            )

        def gath(k, p):
            g = k // GRP
            return pltpu.make_async_copy(
                x_hbm.at[srcv.at[g % 2, k % GRP]], rows.at[p], gsem.at[p])

        def scat(k, p):
            g = k // GRP
            return pltpu.make_async_copy(
                rows.at[p], acc_sh.at[dstv.at[g % 2, k % GRP]], ssem.at[p])

        for d in stage(0, 0):
            d.start()
            d.wait()
        gath(0, 0).start()

        def step(k, carry):
            p = lax.rem(k, 2)
            q = 1 - p
            g = k // GRP
            r = lax.rem(k, GRP)

            @pl.when(k >= 1)
            def _():
                scat(k - 1, q).wait()

            @pl.when(jnp.logical_and(r == 0, g < NGRP - 1))
            def _():
                for d in stage(g + 1, (g + 1) % 2):
                    d.start()

            @pl.when(k < KCH - 1)
            def _():
                @pl.when(r == GRP - 1)
                def _():
                    for d in stage(g + 1, (g + 1) % 2):
                        d.wait()

                gath(k + 1, q).start()

            gath(k, p).wait()
            scat(k, p).start(add=True)

            return carry

        lax.fori_loop(0, KCH, step, 0)
        scat(KCH - 1, (KCH - 1) % 2).wait()

        plsc.subcore_barrier()

        _per_tile_drain(acc_sh, acc_out, c, s)

    return body


_sc_agg_128 = _make_sc_agg(D_IN)


def _make_sc_deg():
    mesh = plsc.VectorSubcoreMesh(core_axis_name="c", subcore_axis_name="s")

    @functools.partial(
        pl.kernel, mesh=mesh,
        out_type=jax.ShapeDtypeStruct((NC, N_NODES, D_IN), jnp.float32),
        scratch_types=[
            pltpu.VMEM((2, GRP, CHUNK), jnp.int32),
            pltpu.VMEM((CHUNK, D_IN), jnp.float32),
            pltpu.VMEM_SHARED((N_NODES, D_IN), jnp.float32),
            pltpu.SemaphoreType.DMA((2,)),
            pltpu.SemaphoreType.DMA,
        ],
    )
    def body(dst_hbm, ones_hbm, zrow_hbm, deg_out, dstv, ones, deg_sh,
             ssem, isem):
        c = lax.axis_index("c")
        s = lax.axis_index("s")
        wid = c * NS + s

        pltpu.sync_copy(ones_hbm, ones)
        _per_tile_zero(zrow_hbm, deg_sh, s)

        plsc.subcore_barrier()

        def stage(g):
            return pltpu.make_async_copy(
                dst_hbm.at[wid, pl.ds(g * GRP, GRP)], dstv.at[g % 2], isem)

        def scat(k, p):
            g = k // GRP
            return pltpu.make_async_copy(
                ones, deg_sh.at[dstv.at[g % 2, k % GRP]], ssem.at[p])

        d = stage(0)
        d.start()
        d.wait()

        def step(k, carry):
            p = lax.rem(k, 2)
            g = k // GRP
            r = lax.rem(k, GRP)

            @pl.when(k >= 1)
            def _():
                scat(k - 1, 1 - p).wait()

            @pl.when(jnp.logical_and(r == 0, g < NGRP - 1))
            def _():
                stage(g + 1).start()

            @pl.when(jnp.logical_and(r == GRP - 1, g < NGRP - 1))
            def _():
                stage(g + 1).wait()

            scat(k, p).start(add=True)

            return carry

        lax.fori_loop(0, KCH, step, 0)
        scat(KCH - 1, (KCH - 1) % 2).wait()

        plsc.subcore_barrier()

        _per_tile_drain(deg_sh, deg_out, c, s)

    return body


_sc_deg = _make_sc_deg()


def _dot_t(a, w):
    return lax.dot_general(a, w, (((1,), (1,)), ((), ())),
                           preferred_element_type=jnp.float32)


def _l2n(v):
    n = jnp.sqrt(jnp.sum(v * v, axis=1, keepdims=True))
    return v / jnp.maximum(n, 1e-12)


def _tc1_body(x_ref, acc_ref, deg_ref, wl_ref, bl_ref, wr_ref, h_ref):
    a = acc_ref[0] + acc_ref[1]
    dg = deg_ref[0, :, 0:1] + deg_ref[1, :, 0:1]
    agg = a / jnp.maximum(dg, 1.0)
    out = _dot_t(agg, wl_ref[...]) + bl_ref[...] + _dot_t(x_ref[...], wr_ref[...])
    h_ref[...] = jnp.maximum(_l2n(out), 0.0)


def _tc2_body(h_ref, acc_ref, deg_ref, batch_ref, wl_ref, bl_ref, wr_ref,
              wlin_ref, blin_ref, out_ref, gsum, gcnt):
    i = pl.program_id(0)

    @pl.when(i == 0)
    def _():
        gsum[...] = jnp.zeros_like(gsum)
        gcnt[...] = jnp.zeros_like(gcnt)

    a = acc_ref[0] + acc_ref[1]
    dg = deg_ref[0, :, 0:1] + deg_ref[1, :, 0:1]
    agg = a / jnp.maximum(dg, 1.0)
    out = _dot_t(agg, wl_ref[...]) + bl_ref[...] + _dot_t(h_ref[...], wr_ref[...])
    h2 = _l2n(out)

    b = batch_ref[0, 0]
    gid = lax.broadcasted_iota(jnp.int32, (NUM_GRAPHS, BLK), 0)
    mask = (b[None, :] == gid).astype(jnp.float32)
    gsum[...] += jnp.dot(mask, h2, preferred_element_type=jnp.float32)
    gcnt[...] += jnp.sum(mask, axis=1, keepdims=True)

    @pl.when(i == NB - 1)
    def _():
        g = gsum[...] / jnp.maximum(gcnt[...], 1.0)
        logits = _dot_t(wlin_ref[...], g) + blin_ref[0, 0]
        out_ref[...] = jax.nn.sigmoid(logits)


_tc1 = pl.pallas_call(
    _tc1_body,
    grid=(NB,),
    in_specs=[
        pl.BlockSpec((BLK, D_IN), lambda i: (i, 0)),
        pl.BlockSpec((NC, BLK, D_IN), lambda i: (0, i, 0)),
        pl.BlockSpec((NC, BLK, D_IN), lambda i: (0, i, 0)),
        pl.BlockSpec((HID, D_IN), lambda i: (0, 0)),
        pl.BlockSpec((1, HID), lambda i: (0, 0)),
        pl.BlockSpec((HID, D_IN), lambda i: (0, 0)),
    ],
    out_specs=pl.BlockSpec((BLK, HID), lambda i: (i, 0)),
    out_shape=jax.ShapeDtypeStruct((N_NODES, HID), jnp.float32),
)

_tc2 = pl.pallas_call(
    _tc2_body,
    grid=(NB,),
    in_specs=[
        pl.BlockSpec((BLK, HID), lambda i: (i, 0)),
        pl.BlockSpec((NC, BLK, HID), lambda i: (0, i, 0)),
        pl.BlockSpec((NC, BLK, D_IN), lambda i: (0, i, 0)),
        pl.BlockSpec((1, 1, BLK), lambda i: (i, 0, 0)),
        pl.BlockSpec((2 * HID, HID), lambda i: (0, 0)),
        pl.BlockSpec((1, 2 * HID), lambda i: (0, 0)),
        pl.BlockSpec((2 * HID, HID), lambda i: (0, 0)),
        pl.BlockSpec((1, 2 * HID), lambda i: (0, 0)),
        pl.BlockSpec((1, 1), lambda i: (0, 0)),
    ],
    out_specs=pl.BlockSpec((1, NUM_GRAPHS), lambda i: (0, 0)),
    out_shape=jax.ShapeDtypeStruct((1, NUM_GRAPHS), jnp.float32),
    scratch_shapes=[
        pltpu.VMEM((NUM_GRAPHS, 2 * HID), jnp.float32),
        pltpu.VMEM((NUM_GRAPHS, 1), jnp.float32),
    ],
)


def kernel(x, edge_index, batch, Wl1, bl1, Wr1, Wl2, bl2, Wr2, Wlin, blin):
    src = edge_index[0].astype(jnp.int32).reshape(NW, KCH, CHUNK)
    dst = edge_index[1].astype(jnp.int32).reshape(NW, KCH, CHUNK)
    batch3 = batch.astype(jnp.int32).reshape(NB, 1, BLK)
    zrow = jnp.zeros((ROWS_LAST, D_IN), jnp.float32)
    ones = jnp.ones((CHUNK, D_IN), jnp.float32)

    deg = _sc_deg(dst, ones, zrow)
    acc1 = _sc_agg_128(x, src, dst, zrow)
    h = _tc1(x, acc1, deg, Wl1, bl1.reshape(1, HID), Wr1)
    acc2 = _sc_agg_128(h, src, dst, zrow)
    out = _tc2(h, acc2, deg, batch3, Wl2, bl2.reshape(1, 2 * HID), Wr2,
               Wlin, blin.reshape(1, 1))
    return out.reshape(NUM_GRAPHS, 1)
